# baseline (device time: 34711 ns/iter reference)
import jax
import jax.numpy as jnp
from jax import lax
from jax.experimental import pallas as pl
from jax.experimental.pallas import tpu as pltpu

N_DEV = 4
N_LAYERS = 3
R = 2


def kernel(x, Win0, Wout0, Win1, Wout1, Win2, Wout2):
    b, d = x.shape
    ch = b // R

    def body(x_ref, win0_ref, wout0_ref, win1_ref, wout1_ref, win2_ref,
             wout2_ref, out_ref, comm_ref, p_ref, xn_ref, send_sems,
             recv_sems):
        my_pos = lax.axis_index("i")

        barrier_sem = pltpu.get_barrier_semaphore()
        for k in range(1, N_DEV):
            pl.semaphore_signal(
                barrier_sem, inc=1,
                device_id=((my_pos + k) % N_DEV,),
                device_id_type=pl.DeviceIdType.MESH,
            )
        pl.semaphore_wait(barrier_sem, N_DEV - 1)

        wins = [win0_ref, win1_ref, win2_ref]
        wouts = [wout0_ref, wout1_ref, wout2_ref]
        sends = {}

        def start_sends(l, r):
            for k in (2, 1, 3):
                dsc = pltpu.make_async_remote_copy(
                    src_ref=p_ref.at[l, r],
                    dst_ref=comm_ref.at[l, r, N_DEV - 1 - k],
                    send_sem=send_sems.at[l, r, k - 1],
                    recv_sem=recv_sems.at[l, r, N_DEV - 1 - k],
                    device_id=((my_pos + k) % N_DEV,),
                    device_id_type=pl.DeviceIdType.MESH,
                )
                dsc.start()
                sends[(l, r, k)] = dsc

        def finish_chunk(l, r):
            acc = p_ref[l, r]
            for m in range(N_DEV - 1):
                recv = pltpu.make_async_remote_copy(
                    src_ref=p_ref.at[l, r],
                    dst_ref=comm_ref.at[l, r, m],
                    send_sem=send_sems.at[l, r, 0],
                    recv_sem=recv_sems.at[l, r, m],
                    device_id=(my_pos,),
                    device_id_type=pl.DeviceIdType.MESH,
                )
                recv.wait_recv()
                acc = acc + comm_ref[l, r, m]
            if l == N_LAYERS - 1:
                out_ref[pl.ds(r * ch, ch), :] = acc
            else:
                xn_ref[l, r] = acc

        for l in range(N_LAYERS):
            for r in range(R):
                if l == 0:
                    xc = x_ref[pl.ds(r * ch, ch), :]
                else:
                    xc = xn_ref[l - 1, r]
                h = jnp.maximum(
                    jnp.dot(xc, wins[l][:, :],
                            preferred_element_type=jnp.float32),
                    0.0,
                )
                p_ref[l, r] = jnp.dot(
                    h, wouts[l][:, :], preferred_element_type=jnp.float32
                )
                start_sends(l, r)
                if r > 0:
                    finish_chunk(l, r - 1)
            finish_chunk(l, R - 1)

        for dsc in sends.values():
            dsc.wait_send()

    return pl.pallas_call(
        body,
        out_shape=jax.ShapeDtypeStruct((b, d), jnp.float32),
        in_specs=[pl.BlockSpec(memory_space=pltpu.VMEM)] * 7,
        out_specs=pl.BlockSpec(memory_space=pltpu.VMEM),
        scratch_shapes=[
            pltpu.VMEM((N_LAYERS, R, N_DEV - 1, ch, d), jnp.float32),
            pltpu.VMEM((N_LAYERS, R, ch, d), jnp.float32),
            pltpu.VMEM((N_LAYERS - 1, R, ch, d), jnp.float32),
            pltpu.SemaphoreType.DMA((N_LAYERS, R, N_DEV - 1)),
            pltpu.SemaphoreType.DMA((N_LAYERS, R, N_DEV - 1)),
        ],
        compiler_params=pltpu.CompilerParams(collective_id=0),
    )(x, Win0, Wout0, Win1, Wout1, Win2, Wout2)


# device time: 32119 ns/iter; 1.0807x vs baseline; 1.0807x over previous
import jax
import jax.numpy as jnp
from jax import lax
from jax.experimental import pallas as pl
from jax.experimental.pallas import tpu as pltpu

N_DEV = 4
N_LAYERS = 3
R = 2


def kernel(x, Win0, Wout0, Win1, Wout1, Win2, Wout2):
    b, d = x.shape
    ch = b // R

    def body(x_ref, win0_ref, wout0_ref, win1_ref, wout1_ref, win2_ref,
             wout2_ref, out_ref, comm_a, comm_b, p_ref, sa_ref,
             send_a, recv_a, send_b, recv_b):
        my_pos = lax.axis_index("i")
        partner_a = my_pos ^ 1
        partner_b = 3 - my_pos

        barrier_sem = pltpu.get_barrier_semaphore()
        for pid in (partner_a, partner_b):
            pl.semaphore_signal(
                barrier_sem, inc=1,
                device_id=(pid,), device_id_type=pl.DeviceIdType.MESH,
            )
        pl.semaphore_wait(barrier_sem, 2)

        wins = [win0_ref, win1_ref, win2_ref]
        wouts = [wout0_ref, wout1_ref, wout2_ref]
        rdma_a = {}
        rdma_b = {}

        def start_a(l, r):
            dsc = pltpu.make_async_remote_copy(
                src_ref=p_ref.at[l, r],
                dst_ref=comm_a.at[l, r],
                send_sem=send_a.at[l, r],
                recv_sem=recv_a.at[l, r],
                device_id=(partner_a,),
                device_id_type=pl.DeviceIdType.MESH,
            )
            dsc.start()
            rdma_a[(l, r)] = dsc

        def finish_a_start_b(l, r):
            rdma_a[(l, r)].wait_recv()
            sa_ref[l, r] = p_ref[l, r] + comm_a[l, r]
            dsc = pltpu.make_async_remote_copy(
                src_ref=sa_ref.at[l, r],
                dst_ref=comm_b.at[l, r],
                send_sem=send_b.at[l, r],
                recv_sem=recv_b.at[l, r],
                device_id=(partner_b,),
                device_id_type=pl.DeviceIdType.MESH,
            )
            dsc.start()
            rdma_b[(l, r)] = dsc

        for l in range(N_LAYERS):
            for r in range(R):
                if l == 0:
                    xc = x_ref[pl.ds(r * ch, ch), :]
                else:
                    rdma_b[(l - 1, r)].wait_recv()
                    xc = sa_ref[l - 1, r] + comm_b[l - 1, r]
                h = jnp.maximum(
                    jnp.dot(xc, wins[l][:, :],
                            preferred_element_type=jnp.float32),
                    0.0,
                )
                p_ref[l, r] = jnp.dot(
                    h, wouts[l][:, :], preferred_element_type=jnp.float32
                )
                start_a(l, r)
                if r > 0:
                    finish_a_start_b(l, r - 1)
            finish_a_start_b(l, R - 1)

        for r in range(R):
            rdma_b[(N_LAYERS - 1, r)].wait_recv()
            out_ref[pl.ds(r * ch, ch), :] = (
                sa_ref[N_LAYERS - 1, r] + comm_b[N_LAYERS - 1, r]
            )

        for dsc in list(rdma_a.values()) + list(rdma_b.values()):
            dsc.wait_send()

    return pl.pallas_call(
        body,
        out_shape=jax.ShapeDtypeStruct((b, d), jnp.float32),
        in_specs=[pl.BlockSpec(memory_space=pltpu.VMEM)] * 7,
        out_specs=pl.BlockSpec(memory_space=pltpu.VMEM),
        scratch_shapes=[
            pltpu.VMEM((N_LAYERS, R, ch, d), jnp.float32),
            pltpu.VMEM((N_LAYERS, R, ch, d), jnp.float32),
            pltpu.VMEM((N_LAYERS, R, ch, d), jnp.float32),
            pltpu.VMEM((N_LAYERS, R, ch, d), jnp.float32),
            pltpu.SemaphoreType.DMA((N_LAYERS, R)),
            pltpu.SemaphoreType.DMA((N_LAYERS, R)),
            pltpu.SemaphoreType.DMA((N_LAYERS, R)),
            pltpu.SemaphoreType.DMA((N_LAYERS, R)),
        ],
        compiler_params=pltpu.CompilerParams(collective_id=0),
    )(x, Win0, Wout0, Win1, Wout1, Win2, Wout2)


# device time: 27257 ns/iter; 1.2735x vs baseline; 1.1784x over previous
import jax
import jax.numpy as jnp
from jax import lax
from jax.experimental import pallas as pl
from jax.experimental.pallas import tpu as pltpu

N_DEV = 4
N_LAYERS = 3
R = 2


def kernel(x, Win0, Wout0, Win1, Wout1, Win2, Wout2):
    b, d = x.shape
    ch = b // R
    hd = d // 2

    def body(x_ref, win0_ref, wout0_ref, win1_ref, wout1_ref, win2_ref,
             wout2_ref, out_ref,
             p1_ref, p2_ref, c1a, c2b, s1_ref, s2_ref, c1b, c2a,
             sp1, rp1, sp2, rp2, ss1, rs1, ss2, rs2):
        my_pos = lax.axis_index("i")
        partner_a = my_pos ^ 1
        partner_b = 3 - my_pos

        barrier_sem = pltpu.get_barrier_semaphore()
        for pid in (partner_a, partner_b):
            pl.semaphore_signal(
                barrier_sem, inc=1,
                device_id=(pid,), device_id_type=pl.DeviceIdType.MESH,
            )
        pl.semaphore_wait(barrier_sem, 2)

        wins = [win0_ref, win1_ref, win2_ref]
        wouts = [wout0_ref, wout1_ref, wout2_ref]
        ph1 = {}
        ph2 = {}

        def mk(src, dst, ssem, rsem, pid):
            return pltpu.make_async_remote_copy(
                src_ref=src, dst_ref=dst, send_sem=ssem, recv_sem=rsem,
                device_id=(pid,), device_id_type=pl.DeviceIdType.MESH,
            )

        def start_phase1(l, r):
            d1 = mk(p1_ref.at[l, r], c1a.at[l, r], sp1.at[l, r],
                    rp1.at[l, r], partner_a)
            d2 = mk(p2_ref.at[l, r], c2b.at[l, r], sp2.at[l, r],
                    rp2.at[l, r], partner_b)
            d1.start()
            d2.start()
            ph1[(l, r)] = (d1, d2)

        def finish_phase1_start_phase2(l, r):
            d1, d2 = ph1[(l, r)]
            d1.wait_recv()
            s1_ref[l, r] = p1_ref[l, r] + c1a[l, r]
            e1 = mk(s1_ref.at[l, r], c1b.at[l, r], ss1.at[l, r],
                    rs1.at[l, r], partner_b)
            e1.start()
            d2.wait_recv()
            s2_ref[l, r] = p2_ref[l, r] + c2b[l, r]
            e2 = mk(s2_ref.at[l, r], c2a.at[l, r], ss2.at[l, r],
                    rs2.at[l, r], partner_a)
            e2.start()
            ph2[(l, r)] = (e1, e2)

        for l in range(N_LAYERS):
            for r in range(R):
                if l == 0:
                    xc = x_ref[pl.ds(r * ch, ch), :]
                    h = jnp.dot(xc, wins[l][:, :],
                                preferred_element_type=jnp.float32)
                else:
                    e1, e2 = ph2[(l - 1, r)]
                    e1.wait_recv()
                    xh1 = s1_ref[l - 1, r] + c1b[l - 1, r]
                    e2.wait_recv()
                    xh2 = s2_ref[l - 1, r] + c2a[l - 1, r]
                    h = (
                        jnp.dot(xh1, wins[l][0:hd, :],
                                preferred_element_type=jnp.float32)
                        + jnp.dot(xh2, wins[l][hd:d, :],
                                  preferred_element_type=jnp.float32)
                    )
                h = jnp.maximum(h, 0.0)
                p = jnp.dot(h, wouts[l][:, :],
                            preferred_element_type=jnp.float32)
                p1_ref[l, r] = p[:, 0:hd]
                p2_ref[l, r] = p[:, hd:d]
                start_phase1(l, r)
                if r > 0:
                    finish_phase1_start_phase2(l, r - 1)
            finish_phase1_start_phase2(l, R - 1)

        L = N_LAYERS - 1
        for r in range(R):
            e1, e2 = ph2[(L, r)]
            e1.wait_recv()
            out_ref[pl.ds(r * ch, ch), 0:hd] = s1_ref[L, r] + c1b[L, r]
            e2.wait_recv()
            out_ref[pl.ds(r * ch, ch), hd:d] = s2_ref[L, r] + c2a[L, r]

        for d1, d2 in list(ph1.values()) + list(ph2.values()):
            d1.wait_send()
            d2.wait_send()

    half = pltpu.VMEM((N_LAYERS, R, ch, hd), jnp.float32)
    sem = pltpu.SemaphoreType.DMA((N_LAYERS, R))
    return pl.pallas_call(
        body,
        out_shape=jax.ShapeDtypeStruct((b, d), jnp.float32),
        in_specs=[pl.BlockSpec(memory_space=pltpu.VMEM)] * 7,
        out_specs=pl.BlockSpec(memory_space=pltpu.VMEM),
        scratch_shapes=[half] * 8 + [sem] * 8,
        compiler_params=pltpu.CompilerParams(collective_id=0),
    )(x, Win0, Wout0, Win1, Wout1, Win2, Wout2)


# device time: 22649 ns/iter; 1.5326x vs baseline; 1.2035x over previous
import jax
import jax.numpy as jnp
from jax import lax
from jax.experimental import pallas as pl
from jax.experimental.pallas import tpu as pltpu

N_DEV = 4
N_LAYERS = 3
R = 2


def kernel(x, Win0, Wout0, Win1, Wout1, Win2, Wout2):
    b, d = x.shape
    ch = b // R
    hd = d // 2

    def body(x_ref, win_ref, wout_ref, out_ref,
             p1_ref, p2_ref, c1a, c2b, s1_ref, s2_ref, c1b, c2a,
             sp1, rp1, sp2, rp2, ss1, rs1, ss2, rs2):
        my_pos = lax.axis_index("i")
        partner_a = my_pos ^ 1
        partner_b = 3 - my_pos

        barrier_sem = pltpu.get_barrier_semaphore()
        for pid in (partner_a, partner_b):
            pl.semaphore_signal(
                barrier_sem, inc=1,
                device_id=(pid,), device_id_type=pl.DeviceIdType.MESH,
            )

        wins = [win_ref.at[i] for i in range(N_LAYERS)]
        wouts = [wout_ref.at[i] for i in range(N_LAYERS)]
        ph1 = {}
        ph2 = {}

        def mk(src, dst, ssem, rsem, pid):
            return pltpu.make_async_remote_copy(
                src_ref=src, dst_ref=dst, send_sem=ssem, recv_sem=rsem,
                device_id=(pid,), device_id_type=pl.DeviceIdType.MESH,
            )

        def start_phase1(l, r):
            d1 = mk(p1_ref.at[l, r], c1a.at[l, r], sp1.at[l, r],
                    rp1.at[l, r], partner_a)
            d2 = mk(p2_ref.at[l, r], c2b.at[l, r], sp2.at[l, r],
                    rp2.at[l, r], partner_b)
            d1.start()
            d2.start()
            ph1[(l, r)] = (d1, d2)

        def finish_phase1_start_phase2(l, r):
            d1, d2 = ph1[(l, r)]
            d1.wait_recv()
            s1_ref[l, r] = p1_ref[l, r] + c1a[l, r]
            e1 = mk(s1_ref.at[l, r], c1b.at[l, r], ss1.at[l, r],
                    rs1.at[l, r], partner_b)
            e1.start()
            d2.wait_recv()
            s2_ref[l, r] = p2_ref[l, r] + c2b[l, r]
            e2 = mk(s2_ref.at[l, r], c2a.at[l, r], ss2.at[l, r],
                    rs2.at[l, r], partner_a)
            e2.start()
            ph2[(l, r)] = (e1, e2)

        for l in range(N_LAYERS):
            for r in range(R):
                if l == 0:
                    xc = x_ref[pl.ds(r * ch, ch), :]
                    h = jnp.dot(xc, wins[l][:, :],
                                preferred_element_type=jnp.float32)
                else:
                    e1, e2 = ph2[(l - 1, r)]
                    e1.wait_recv()
                    xh1 = s1_ref[l - 1, r] + c1b[l - 1, r]
                    h1 = jnp.dot(xh1, wins[l][0:hd, :],
                                 preferred_element_type=jnp.float32)
                    e2.wait_recv()
                    xh2 = s2_ref[l - 1, r] + c2a[l - 1, r]
                    h = h1 + jnp.dot(xh2, wins[l][hd:d, :],
                                     preferred_element_type=jnp.float32)
                h = jnp.maximum(h, 0.0)
                p = jnp.dot(h, wouts[l][:, :],
                            preferred_element_type=jnp.float32)
                p1_ref[l, r] = p[:, 0:hd]
                p2_ref[l, r] = p[:, hd:d]
                if l == 0 and r == 0:
                    pl.semaphore_wait(barrier_sem, 2)
                start_phase1(l, r)
                if r > 0:
                    finish_phase1_start_phase2(l, r - 1)
            finish_phase1_start_phase2(l, R - 1)

        L = N_LAYERS - 1
        for r in range(R):
            e1, e2 = ph2[(L, r)]
            e1.wait_recv()
            out_ref[pl.ds(r * ch, ch), 0:hd] = s1_ref[L, r] + c1b[L, r]
            e2.wait_recv()
            out_ref[pl.ds(r * ch, ch), hd:d] = s2_ref[L, r] + c2a[L, r]

        for d1, d2 in list(ph1.values()) + list(ph2.values()):
            d1.wait_send()
            d2.wait_send()

    half = pltpu.VMEM((N_LAYERS, R, ch, hd), jnp.float32)
    sem = pltpu.SemaphoreType.DMA((N_LAYERS, R))
    return pl.pallas_call(
        body,
        out_shape=jax.ShapeDtypeStruct((b, d), jnp.float32),
        in_specs=[pl.BlockSpec(memory_space=pltpu.VMEM)] * 3,
        out_specs=pl.BlockSpec(memory_space=pltpu.VMEM),
        scratch_shapes=[half] * 8 + [sem] * 8,
        compiler_params=pltpu.CompilerParams(collective_id=0),
    )(
        x,
        jnp.stack([Win0, Win1, Win2]),
        jnp.stack([Wout0, Wout1, Wout2]),
    )
